# baseline (speedup 1.0000x reference)
import functools

import jax
import jax.numpy as jnp
from jax import lax
from jax.experimental import pallas as pl
from jax.experimental.pallas import tpu as pltpu
from jax.experimental.pallas import tpu_sc as plsc

_NUM_ATOMS = 262144
_TABLE_ROWS = 128
_OUT_DIM = 128

_info = plsc.get_sparse_core_info()
_NC = _info.num_cores
_NS = _info.num_subcores
_NW = _NC * _NS
_B_PER_W = _NUM_ATOMS // _NW
_SG = 256
_N_SG = _B_PER_W // _SG
_LANES = 16
_NBLK = _SG // _LANES


def _project_body(fm_ref, w_ref, b_ref, out_ref):
    out_ref[...] = lax.dot_general(
        fm_ref[...], w_ref[...], (((1,), (1,)), ((), ())),
        preferred_element_type=jnp.float32) + b_ref[...]


def _project(feature_map, W, b):
    return pl.pallas_call(
        _project_body,
        out_shape=jax.ShapeDtypeStruct((_TABLE_ROWS, _OUT_DIM), jnp.float32),
    )(feature_map, W, b.reshape(1, _OUT_DIM))


_mesh = plsc.VectorSubcoreMesh(core_axis_name="c", subcore_axis_name="s")


@functools.partial(
    pl.kernel,
    mesh=_mesh,
    compiler_params=pltpu.CompilerParams(needs_layout_passes=False),
    out_type=jax.ShapeDtypeStruct((_NUM_ATOMS * _OUT_DIM,), jnp.float32),
    scratch_types=[
        pltpu.VMEM((_TABLE_ROWS * _OUT_DIM,), jnp.float32),
        pltpu.VMEM((_B_PER_W,), jnp.int32),
        pltpu.VMEM((2, _SG * _OUT_DIM), jnp.float32),
        pltpu.SemaphoreType.DMA,
        pltpu.SemaphoreType.DMA,
    ],
)
def _expand(table_hbm, idx_hbm, out_hbm, table_v, idx_v, rows_v, w0, w1):
    wid = lax.axis_index("s") * _NC + lax.axis_index("c")
    base = wid * _B_PER_W * _OUT_DIM
    wsems = (w0, w1)
    pltpu.sync_copy(table_hbm, table_v)
    pltpu.sync_copy(idx_hbm.at[wid], idx_v)
    lane_off = lax.iota(jnp.int32, _LANES) * _OUT_DIM

    def group(p, q, wait_write):
        if wait_write:
            pltpu.make_async_copy(
                rows_v.at[q], out_hbm.at[pl.ds(base, _SG * _OUT_DIM)],
                wsems[q]).wait()
        q_vec = jnp.full((_LANES,), q, jnp.int32)

        def blk(t, carry):
            vi = idx_v[pl.ds(p * _SG + t * _LANES, _LANES)]
            src_base = vi * _OUT_DIM
            dst_base = lane_off + t * (_LANES * _OUT_DIM)
            for c in range(_OUT_DIM):
                vals = plsc.load_gather(table_v, [src_base + c])
                plsc.store_scatter(rows_v, [q_vec, dst_base + c], vals)
            return carry

        lax.fori_loop(0, _NBLK, blk, 0)
        pltpu.async_copy(
            rows_v.at[q],
            out_hbm.at[pl.ds(base + p * _SG * _OUT_DIM, _SG * _OUT_DIM)],
            wsems[q])

    group(0, 0, wait_write=False)
    group(1, 1, wait_write=False)

    def body(gg, carry):
        group(2 * gg, 0, wait_write=True)
        group(2 * gg + 1, 1, wait_write=True)
        return carry

    lax.fori_loop(1, _N_SG // 2, body, 0)

    for q in range(2):
        pltpu.make_async_copy(
            rows_v.at[q], out_hbm.at[pl.ds(base, _SG * _OUT_DIM)],
            wsems[q]).wait()


def kernel(src, feature_map, W, b):
    proj = _project(feature_map, W, b)
    idx = src.astype(jnp.int32).reshape(_NW, _B_PER_W)
    out = _expand(proj.reshape(-1), idx)
    return out.reshape(_NUM_ATOMS, _OUT_DIM)


# speedup vs baseline: 4.7369x; 4.7369x over previous
import functools

import jax
import jax.numpy as jnp
from jax import lax
from jax.experimental import pallas as pl
from jax.experimental.pallas import tpu as pltpu
from jax.experimental.pallas import tpu_sc as plsc

_NUM_ATOMS = 262144
_TABLE_ROWS = 128
_OUT_DIM = 128

_info = plsc.get_sparse_core_info()
_NC = _info.num_cores
_NS = _info.num_subcores
_NW = _NC * _NS
_B_PER_W = _NUM_ATOMS // _NW
_SG = 256
_N_SG = _B_PER_W // _SG
_LANES = 16
_NBLK = _SG // _LANES
_CVEC = _OUT_DIM // _LANES


def _project_body(fm_ref, w_ref, b_ref, out_ref):
    out_ref[...] = lax.dot_general(
        fm_ref[...], w_ref[...], (((1,), (1,)), ((), ())),
        preferred_element_type=jnp.float32) + b_ref[...]


def _project(feature_map, W, b):
    return pl.pallas_call(
        _project_body,
        out_shape=jax.ShapeDtypeStruct((_TABLE_ROWS, _OUT_DIM), jnp.float32),
    )(feature_map, W, b.reshape(1, _OUT_DIM))


_mesh = plsc.VectorSubcoreMesh(core_axis_name="c", subcore_axis_name="s")


@functools.partial(
    pl.kernel,
    mesh=_mesh,
    compiler_params=pltpu.CompilerParams(needs_layout_passes=False),
    out_type=jax.ShapeDtypeStruct((_NUM_ATOMS, _OUT_DIM), jnp.float32),
    scratch_types=[
        pltpu.VMEM((_TABLE_ROWS, _OUT_DIM), jnp.float32),
        pltpu.VMEM((_B_PER_W,), jnp.int32),
        pltpu.VMEM((2, _SG, _OUT_DIM), jnp.float32),
        pltpu.SemaphoreType.DMA,
        pltpu.SemaphoreType.DMA,
    ],
)
def _expand(table_hbm, idx_hbm, out_hbm, table_v, idx_v, rows_v, w0, w1):
    wid = lax.axis_index("s") * _NC + lax.axis_index("c")
    base = wid * _B_PER_W
    wsems = (w0, w1)
    pltpu.sync_copy(table_hbm, table_v)
    pltpu.sync_copy(idx_hbm.at[wid], idx_v)

    def group(p, q, wait_write):
        if wait_write:
            pltpu.make_async_copy(
                rows_v.at[q], out_hbm.at[pl.ds(base, _SG)], wsems[q]).wait()

        def blk(t, carry):
            vi = idx_v[pl.ds(p * _SG + t * _LANES, _LANES)]
            for l in range(_LANES):
                r = vi[l]
                row = t * _LANES + l
                for c in range(_CVEC):
                    rows_v[q, row, pl.ds(c * _LANES, _LANES)] = (
                        table_v[r, pl.ds(c * _LANES, _LANES)])
            return carry

        lax.fori_loop(0, _NBLK, blk, 0)
        pltpu.async_copy(
            rows_v.at[q], out_hbm.at[pl.ds(base + p * _SG, _SG)], wsems[q])

    group(0, 0, wait_write=False)
    group(1, 1, wait_write=False)

    def body(gg, carry):
        group(2 * gg, 0, wait_write=True)
        group(2 * gg + 1, 1, wait_write=True)
        return carry

    lax.fori_loop(1, _N_SG // 2, body, 0)

    for q in range(2):
        pltpu.make_async_copy(
            rows_v.at[q], out_hbm.at[pl.ds(base, _SG)], wsems[q]).wait()


def kernel(src, feature_map, W, b):
    proj = _project(feature_map, W, b)
    idx = src.astype(jnp.int32).reshape(_NW, _B_PER_W)
    return _expand(proj, idx)


# speedup vs baseline: 16.8170x; 3.5502x over previous
import functools

import jax
import jax.numpy as jnp
from jax import lax
from jax.experimental import pallas as pl
from jax.experimental.pallas import tpu as pltpu
from jax.experimental.pallas import tpu_sc as plsc

_NUM_ATOMS = 262144
_TABLE_ROWS = 128
_OUT_DIM = 128

_info = plsc.get_sparse_core_info()
_NC = _info.num_cores
_NS = _info.num_subcores
_NW = _NC * _NS
_B_PER_W = _NUM_ATOMS // _NW
_CHUNK = 128
_N_CHUNKS = _B_PER_W // _CHUNK
_G = 2
_SG = _G * _CHUNK
_N_SG = _B_PER_W // _SG


def _project_body(fm_ref, w_ref, b_ref, out_ref):
    out_ref[...] = lax.dot_general(
        fm_ref[...], w_ref[...], (((1,), (1,)), ((), ())),
        preferred_element_type=jnp.float32) + b_ref[...]


def _project(feature_map, W, b):
    return pl.pallas_call(
        _project_body,
        out_shape=jax.ShapeDtypeStruct((_TABLE_ROWS, _OUT_DIM), jnp.float32),
    )(feature_map, W, b.reshape(1, _OUT_DIM))


_mesh = plsc.VectorSubcoreMesh(core_axis_name="c", subcore_axis_name="s")


@functools.partial(
    pl.kernel,
    mesh=_mesh,
    out_type=jax.ShapeDtypeStruct((_NUM_ATOMS, _OUT_DIM), jnp.float32),
    scratch_types=[
        pltpu.VMEM_SHARED((_TABLE_ROWS, _OUT_DIM), jnp.float32),
        pltpu.VMEM((_N_CHUNKS, _CHUNK), jnp.int32),
        pltpu.VMEM((2, _SG, _OUT_DIM), jnp.float32),
        pltpu.SemaphoreType.DMA,
        pltpu.SemaphoreType.DMA,
        pltpu.SemaphoreType.DMA,
        pltpu.SemaphoreType.DMA,
    ],
)
def _gather(table_hbm, idx_hbm, out_hbm, table_s, idx_v, rows_v, g0, g1, w0, w1):
    sid = lax.axis_index("s")
    wid = sid * _NC + lax.axis_index("c")
    base = wid * _B_PER_W
    gsems = (g0, g1)
    wsems = (w0, w1)

    @pl.when(sid == 0)
    def _stage():
        pltpu.sync_copy(table_hbm, table_s)

    plsc.subcore_barrier()
    pltpu.sync_copy(idx_hbm.at[wid], idx_v)

    def group(p, q, wait_write):
        if wait_write:
            pltpu.make_async_copy(
                rows_v.at[q], out_hbm.at[pl.ds(base, _SG)], wsems[q]).wait()
        handles = [
            pltpu.async_copy(
                table_s.at[idx_v.at[p * _G + k]],
                rows_v.at[q, pl.ds(k * _CHUNK, _CHUNK)],
                gsems[q])
            for k in range(_G)
        ]
        for h in handles:
            h.wait()
        pltpu.async_copy(
            rows_v.at[q], out_hbm.at[pl.ds(base + p * _SG, _SG)], wsems[q])

    group(0, 0, wait_write=False)
    group(1, 1, wait_write=False)

    def body(gg, carry):
        group(2 * gg, 0, wait_write=True)
        group(2 * gg + 1, 1, wait_write=True)
        return carry

    lax.fori_loop(1, _N_SG // 2, body, 0)

    for q in range(2):
        pltpu.make_async_copy(
            rows_v.at[q], out_hbm.at[pl.ds(base, _SG)], wsems[q]).wait()


def kernel(src, feature_map, W, b):
    proj = _project(feature_map, W, b)
    idx = src.astype(jnp.int32).reshape(_NW, _N_CHUNKS, _CHUNK)
    return _gather(proj, idx)


# speedup vs baseline: 17.0084x; 1.0114x over previous
import functools

import jax
import jax.numpy as jnp
from jax import lax
from jax.experimental import pallas as pl
from jax.experimental.pallas import tpu as pltpu
from jax.experimental.pallas import tpu_sc as plsc

_NUM_ATOMS = 262144
_TABLE_ROWS = 128
_OUT_DIM = 128

_info = plsc.get_sparse_core_info()
_NC = _info.num_cores
_NS = _info.num_subcores
_NW = _NC * _NS
_B_PER_W = _NUM_ATOMS // _NW
_CHUNK = 128
_N_CHUNKS = _B_PER_W // _CHUNK
_G = 2
_SG = _G * _CHUNK
_N_SG = _B_PER_W // _SG


def _project_body(fm_ref, w_ref, b_ref, out_ref):
    out_ref[...] = lax.dot_general(
        fm_ref[...], w_ref[...], (((1,), (1,)), ((), ())),
        preferred_element_type=jnp.float32) + b_ref[...]


def _project(feature_map, W, b):
    return pl.pallas_call(
        _project_body,
        out_shape=jax.ShapeDtypeStruct((_TABLE_ROWS, _OUT_DIM), jnp.float32),
    )(feature_map, W, b.reshape(1, _OUT_DIM))


_mesh = plsc.VectorSubcoreMesh(core_axis_name="c", subcore_axis_name="s")


@functools.partial(
    pl.kernel,
    mesh=_mesh,
    out_type=jax.ShapeDtypeStruct((_NUM_ATOMS, _OUT_DIM), jnp.float32),
    scratch_types=[
        pltpu.VMEM_SHARED((_TABLE_ROWS, _OUT_DIM), jnp.float32),
        pltpu.VMEM((_N_CHUNKS, _CHUNK), jnp.int32),
        pltpu.VMEM((2, _SG, _OUT_DIM), jnp.float32),
        pltpu.SemaphoreType.DMA,
        pltpu.SemaphoreType.DMA,
        pltpu.SemaphoreType.DMA,
        pltpu.SemaphoreType.DMA,
    ],
)
def _gather(table_hbm, idx_hbm, out_hbm, table_s, idx_v, rows_v, g0, g1, w0, w1):
    sid = lax.axis_index("s")
    wid = sid * _NC + lax.axis_index("c")
    base = wid * _B_PER_W
    gsems = (g0, g1)
    wsems = (w0, w1)

    idx_copy = pltpu.async_copy(idx_hbm.at[wid], idx_v, w0)

    @pl.when(sid == 0)
    def _stage():
        pltpu.sync_copy(table_hbm, table_s)

    plsc.subcore_barrier()
    idx_copy.wait()

    def group(p, q, wait_write):
        if wait_write:
            pltpu.make_async_copy(
                rows_v.at[q], out_hbm.at[pl.ds(base, _SG)], wsems[q]).wait()
        handles = [
            pltpu.async_copy(
                table_s.at[idx_v.at[p * _G + k]],
                rows_v.at[q, pl.ds(k * _CHUNK, _CHUNK)],
                gsems[q])
            for k in range(_G)
        ]
        for h in handles:
            h.wait()
        pltpu.async_copy(
            rows_v.at[q], out_hbm.at[pl.ds(base + p * _SG, _SG)], wsems[q])

    group(0, 0, wait_write=False)
    group(1, 1, wait_write=False)

    def body(gg, carry):
        group(2 * gg, 0, wait_write=True)
        group(2 * gg + 1, 1, wait_write=True)
        return carry

    lax.fori_loop(1, _N_SG // 2, body, 0)

    for q in range(2):
        pltpu.make_async_copy(
            rows_v.at[q], out_hbm.at[pl.ds(base, _SG)], wsems[q]).wait()


def kernel(src, feature_map, W, b):
    proj = _project(feature_map, W, b)
    idx = src.astype(jnp.int32).reshape(_NW, _N_CHUNKS, _CHUNK)
    return _gather(proj, idx)
